# Initial kernel scaffold; baseline (speedup 1.0000x reference)
import jax
import jax.numpy as jnp
from jax.experimental import pallas as pl


def kernel(species, embedding, idx_H, idx_C, idx_N, idx_O, W1, b1, W2, b2, W3, b3):
    raise NotImplementedError("write your pallas kernel here")


# speedup vs baseline: 2.5785x; 2.5785x over previous
import functools

import jax
import jax.numpy as jnp
from jax import lax
from jax.experimental import pallas as pl
from jax.experimental.pallas import tpu as pltpu
from jax.experimental.pallas import tpu_sc as plsc


def _sc_info():
    info = plsc.get_sparse_core_info()
    return info.num_cores, info.num_subcores


def _sc_gather(table, idx, d):
    nc, ns = _sc_info()
    nw = nc * ns
    b = idx.shape[0]
    per_w = b // nw
    ch = 112
    while per_w % ch:
        ch //= 2
    n_ch = per_w // ch
    mesh = plsc.VectorSubcoreMesh(core_axis_name="c", subcore_axis_name="s")

    @functools.partial(
        pl.kernel,
        mesh=mesh,
        out_type=jax.ShapeDtypeStruct((b, d), jnp.float32),
        scratch_types=[
            pltpu.VMEM((ch,), jnp.int32),
            pltpu.VMEM((ch, d), jnp.float32),
            pltpu.SemaphoreType.DMA,
        ],
    )
    def k(table_hbm, idx_hbm, out_hbm, idx_v, rows_v, sem):
        wid = lax.axis_index("s") * nc + lax.axis_index("c")
        base = wid * per_w

        def body(c, carry):
            off = base + c * ch
            pltpu.sync_copy(idx_hbm.at[pl.ds(off, ch)], idx_v)
            pltpu.async_copy(table_hbm.at[idx_v], rows_v, sem).wait()
            pltpu.sync_copy(rows_v, out_hbm.at[pl.ds(off, ch)])
            return carry

        lax.fori_loop(0, n_ch, body, 0)

    return k(table, idx)


def _sc_scatter(rows, idx, n_out, d):
    nc, ns = _sc_info()
    nw = nc * ns
    b = idx.shape[0]
    per_w = b // nw
    ch = 112
    while per_w % ch:
        ch //= 2
    n_ch = per_w // ch
    mesh = plsc.VectorSubcoreMesh(core_axis_name="c", subcore_axis_name="s")

    @functools.partial(
        pl.kernel,
        mesh=mesh,
        out_type=jax.ShapeDtypeStruct((n_out, d), jnp.float32),
        scratch_types=[
            pltpu.VMEM((ch,), jnp.int32),
            pltpu.VMEM((ch, d), jnp.float32),
            pltpu.SemaphoreType.DMA,
        ],
    )
    def k(rows_hbm, idx_hbm, out_hbm, idx_v, rows_v, sem):
        wid = lax.axis_index("s") * nc + lax.axis_index("c")
        base = wid * per_w

        def body(c, carry):
            off = base + c * ch
            pltpu.sync_copy(idx_hbm.at[pl.ds(off, ch)], idx_v)
            pltpu.sync_copy(rows_hbm.at[pl.ds(off, ch)], rows_v)
            pltpu.async_copy(rows_v, out_hbm.at[idx_v], sem).wait()
            return carry

        lax.fori_loop(0, n_ch, body, 0)

    return k(rows, idx)


def _mlp_body(x_ref, w1_ref, b1_ref, w2_ref, b2_ref, w3_ref, b3_ref, o_ref):
    x = x_ref[...]
    h = jnp.dot(x, w1_ref[0], preferred_element_type=jnp.float32) + b1_ref[0]
    h = h * (1.0 / (1.0 + jnp.exp(-h)))
    h = jnp.dot(h, w2_ref[0], preferred_element_type=jnp.float32) + b2_ref[0]
    h = h * (1.0 / (1.0 + jnp.exp(-h)))
    o_ref[...] = (
        jnp.dot(h, w3_ref[0], preferred_element_type=jnp.float32) + b3_ref[0]
    )


def _mlp(x, W1, b1, W2, b2, W3, b3, n_species, rows_per_species):
    d_in = W1.shape[1]
    d_h = W1.shape[2]
    d_out = W3.shape[2]
    br = 256
    nr = rows_per_species // br
    grid = (n_species, nr)
    b1r = b1.reshape(n_species, 1, d_h)
    b2r = b2.reshape(n_species, 1, d_h)
    b3r = b3.reshape(n_species, 1, d_out)
    return pl.pallas_call(
        _mlp_body,
        grid=grid,
        in_specs=[
            pl.BlockSpec((br, d_in), lambda s, r: (s * nr + r, 0)),
            pl.BlockSpec((1, d_in, d_h), lambda s, r: (s, 0, 0)),
            pl.BlockSpec((1, 1, d_h), lambda s, r: (s, 0, 0)),
            pl.BlockSpec((1, d_h, d_h), lambda s, r: (s, 0, 0)),
            pl.BlockSpec((1, 1, d_h), lambda s, r: (s, 0, 0)),
            pl.BlockSpec((1, d_h, d_out), lambda s, r: (s, 0, 0)),
            pl.BlockSpec((1, 1, d_out), lambda s, r: (s, 0, 0)),
        ],
        out_specs=pl.BlockSpec((br, d_out), lambda s, r: (s * nr + r, 0)),
        out_shape=jax.ShapeDtypeStruct(
            (n_species * rows_per_species, d_out), jnp.float32
        ),
    )(x, W1, b1r, W2, b2r, W3, b3r)


def kernel(species, embedding, idx_H, idx_C, idx_N, idx_O, W1, b1, W2, b2, W3, b3):
    n_atoms = species.shape[0]
    d_in = embedding.shape[1]
    d_out = W3.shape[2]
    n_species = W1.shape[0]
    idxs = [idx_H, idx_C, idx_N, idx_O]
    per = idx_H.shape[0]
    pad_to = -(-per // 256) * 256
    pad = pad_to - per

    idx_g = jnp.concatenate(
        [jnp.pad(i.astype(jnp.int32), (0, pad)) for i in idxs]
    )
    idx_s = jnp.concatenate(
        [
            jnp.pad(i.astype(jnp.int32), (0, pad), constant_values=n_atoms)
            for i in idxs
        ]
    )

    x = _sc_gather(embedding, idx_g, d_in)
    o = _mlp(x, W1, b1, W2, b2, W3, b3, n_species, pad_to)
    out_big = _sc_scatter(o, idx_s, n_species * pad_to, d_out)
    return out_big[:n_atoms]
